# baseline (speedup 1.0000x reference)
import jax
import jax.numpy as jnp
from jax.experimental import pallas as pl
from jax.experimental.pallas import tpu as pltpu

_ROWS = 4096
_D = 1024
_CHUNKS = (1536, 2560)
_OFFS = tuple(sum(_CHUNKS[:i]) for i in range(len(_CHUNKS)))
_N = len(_CHUNKS)


def _copy_body(x_ref, o_ref, vmem, load_sems, store_sems):
    loads = []
    for i in range(_N):
        c = pltpu.make_async_copy(
            x_ref.at[pl.ds(_OFFS[i], _CHUNKS[i]), :],
            vmem.at[pl.ds(_OFFS[i], _CHUNKS[i]), :],
            load_sems.at[i],
        )
        c.start()
        loads.append(c)
    stores = []
    for i in range(_N):
        loads[i].wait()
        c = pltpu.make_async_copy(
            vmem.at[pl.ds(_OFFS[i], _CHUNKS[i]), :],
            o_ref.at[pl.ds(_OFFS[i], _CHUNKS[i]), :],
            store_sems.at[i],
        )
        c.start()
        stores.append(c)
    for c in stores:
        c.wait()


def kernel(x):
    b, s, d = x.shape
    x2 = x.reshape(_ROWS, _D)
    out = pl.pallas_call(
        _copy_body,
        in_specs=[pl.BlockSpec(memory_space=pl.ANY)],
        out_specs=pl.BlockSpec(memory_space=pl.ANY),
        scratch_shapes=[
            pltpu.VMEM((_ROWS, _D), jnp.float32),
            pltpu.SemaphoreType.DMA((_N,)),
            pltpu.SemaphoreType.DMA((_N,)),
        ],
        out_shape=jax.ShapeDtypeStruct((_ROWS, _D), x.dtype),
    )(x2)
    return out.reshape(b, s, d)


# speedup vs baseline: 1.0953x; 1.0953x over previous
import jax
import jax.numpy as jnp
from jax.experimental import pallas as pl
from jax.experimental.pallas import tpu as pltpu

_ROWS = 4096
_D = 1024
_CHUNKS = (2048, 2048)
_OFFS = tuple(sum(_CHUNKS[:i]) for i in range(len(_CHUNKS)))
_N = len(_CHUNKS)


def _copy_body(x_ref, o_ref, vmem, load_sems, store_sems):
    loads = []
    for i in range(_N):
        c = pltpu.make_async_copy(
            x_ref.at[pl.ds(_OFFS[i], _CHUNKS[i]), :],
            vmem.at[pl.ds(_OFFS[i], _CHUNKS[i]), :],
            load_sems.at[i],
        )
        c.start()
        loads.append(c)
    stores = []
    for i in range(_N):
        loads[i].wait()
        c = pltpu.make_async_copy(
            vmem.at[pl.ds(_OFFS[i], _CHUNKS[i]), :],
            o_ref.at[pl.ds(_OFFS[i], _CHUNKS[i]), :],
            store_sems.at[i],
        )
        c.start()
        stores.append(c)
    for c in stores:
        c.wait()


def kernel(x):
    b, s, d = x.shape
    x2 = x.reshape(_ROWS, _D)
    out = pl.pallas_call(
        _copy_body,
        in_specs=[pl.BlockSpec(memory_space=pl.ANY)],
        out_specs=pl.BlockSpec(memory_space=pl.ANY),
        scratch_shapes=[
            pltpu.VMEM((_ROWS, _D), jnp.float32),
            pltpu.SemaphoreType.DMA((_N,)),
            pltpu.SemaphoreType.DMA((_N,)),
        ],
        out_shape=jax.ShapeDtypeStruct((_ROWS, _D), x.dtype),
    )(x2)
    return out.reshape(b, s, d)
